# Initial kernel scaffold; baseline (speedup 1.0000x reference)
import jax
import jax.numpy as jnp
from jax.experimental import pallas as pl


def kernel(x, edge_index, batch, W1, b1, W2, b2, Wlin, blin):
    raise NotImplementedError("write your pallas kernel here")


# speedup vs baseline: 42.0665x; 42.0665x over previous
import functools

import jax
import jax.numpy as jnp
from jax import lax
from jax.experimental import pallas as pl
from jax.experimental.pallas import tpu as pltpu
from jax.experimental.pallas import tpu_sc as plsc

NUM_GRAPHS = 64
DP = 48
HALF = DP // 2
EB = 2000
NB = 2000
NCORES = 2
NSUB = 16


def _sc_degree(dst, n):
    e = dst.shape[0]
    per_core = e // NCORES
    per_tile = per_core // NSUB
    n_iter = per_tile // EB
    assert per_core * NCORES == e and per_tile * NSUB == per_core
    assert n_iter * EB == per_tile and n % NB == 0
    n_chunks = n // NB
    mesh = plsc.VectorSubcoreMesh(core_axis_name="c", subcore_axis_name="s")

    @functools.partial(
        pl.kernel,
        out_type=[jax.ShapeDtypeStruct((n,), jnp.float32)] * 2,
        mesh=mesh,
        scratch_types=[
            pltpu.VMEM((EB,), jnp.int32),
            pltpu.VMEM((EB,), jnp.float32),
            pltpu.VMEM((NB,), jnp.float32),
            pltpu.VMEM_SHARED((n,), jnp.float32),
        ],
    )
    def k(dst_hbm, out0, out1, idx_v, ones_v, zero_v, acc_sh):
        c = lax.axis_index("c")
        s = lax.axis_index("s")

        @pl.loop(0, EB // 16)
        def _(i):
            ones_v[pl.ds(i * 16, 16)] = jnp.ones((16,), jnp.float32)

        @pl.loop(0, NB // 16)
        def _(i):
            zero_v[pl.ds(i * 16, 16)] = jnp.zeros((16,), jnp.float32)

        for j in range(n_chunks):
            @pl.when(s == j % NSUB)
            def _():
                pltpu.sync_copy(zero_v, acc_sh.at[pl.ds(j * NB, NB)])

        plsc.subcore_barrier()

        base0 = c * per_core + s * per_tile

        @pl.loop(0, n_iter)
        def _(i):
            b = base0 + i * EB
            pltpu.sync_copy(dst_hbm.at[pl.ds(b, EB)], idx_v)
            pltpu.sync_copy(ones_v, acc_sh.at[idx_v], add=True)

        plsc.subcore_barrier()

        for j in range(n_chunks):
            @pl.when(s == j % NSUB)
            def _():
                sl = pl.ds(j * NB, NB)
                pltpu.sync_copy(acc_sh.at[sl], zero_v)

                @pl.when(c == 0)
                def _():
                    pltpu.sync_copy(zero_v, out0.at[sl])

                @pl.when(c == 1)
                def _():
                    pltpu.sync_copy(zero_v, out1.at[sl])

    return k(dst)


def _sc_aggregate(src, dst, z_left, z_right, n):
    e = src.shape[0]
    per_tile = e // NSUB
    n_iter = per_tile // EB
    assert per_tile * NSUB == e and n_iter * EB == per_tile
    n_chunks = n // NB
    mesh = plsc.VectorSubcoreMesh(core_axis_name="c", subcore_axis_name="s")

    @functools.partial(
        pl.kernel,
        out_type=[jax.ShapeDtypeStruct((n, HALF), jnp.float32)] * 2,
        mesh=mesh,
        scratch_types=[
            pltpu.VMEM((EB,), jnp.int32),
            pltpu.VMEM((EB,), jnp.int32),
            pltpu.VMEM((EB, HALF), jnp.float32),
            pltpu.VMEM_SHARED((n, HALF), jnp.float32),
            pltpu.SemaphoreType.DMA,
        ],
        compiler_params=pltpu.CompilerParams(use_tc_tiling_on_sc=False),
    )
    def k(src_hbm, dst_hbm, zl_hbm, zr_hbm, out_l, out_r,
          si_v, di_v, rows_v, acc_sh, sem):
        c = lax.axis_index("c")
        s = lax.axis_index("s")

        @pl.loop(0, EB)
        def _(i):
            rows_v[i, pl.ds(0, 16)] = jnp.zeros((16,), jnp.float32)
            rows_v[i, pl.ds(HALF - 16, 16)] = jnp.zeros((16,), jnp.float32)

        for j in range(n_chunks):
            @pl.when(s == j % NSUB)
            def _():
                pltpu.sync_copy(rows_v, acc_sh.at[pl.ds(j * NB, NB)])

        plsc.subcore_barrier()

        base0 = s * per_tile

        @pl.loop(0, n_iter)
        def _(i):
            b = base0 + i * EB
            pltpu.sync_copy(src_hbm.at[pl.ds(b, EB)], si_v)
            pltpu.sync_copy(dst_hbm.at[pl.ds(b, EB)], di_v)

            @pl.when(c == 0)
            def _():
                pltpu.async_copy(zl_hbm.at[si_v], rows_v, sem).wait()

            @pl.when(c == 1)
            def _():
                pltpu.async_copy(zr_hbm.at[si_v], rows_v, sem).wait()

            pltpu.sync_copy(rows_v, acc_sh.at[di_v], add=True)

        plsc.subcore_barrier()

        for j in range(n_chunks):
            @pl.when(s == j % NSUB)
            def _():
                sl = pl.ds(j * NB, NB)
                pltpu.sync_copy(acc_sh.at[sl], rows_v)

                @pl.when(c == 0)
                def _():
                    pltpu.sync_copy(rows_v, out_l.at[sl])

                @pl.when(c == 1)
                def _():
                    pltpu.sync_copy(rows_v, out_r.at[sl])

    return k(src, dst, z_left, z_right)


def _tc_z1(x, w1p, d0, d1, n, bn):
    grid = n // bn
    din = x.shape[1]

    def body(x_ref, w_ref, d0_ref, d1_ref, zl_ref, zr_ref, dinv_ref):
        deg = 1.0 + d0_ref[...] + d1_ref[...]
        dinv = lax.rsqrt(deg)
        xw = jnp.dot(x_ref[...], w_ref[...], preferred_element_type=jnp.float32)
        z = xw * dinv
        zl_ref[...] = z[:, :HALF]
        zr_ref[...] = z[:, HALF:]
        dinv_ref[...] = dinv

    return pl.pallas_call(
        body,
        grid=(grid,),
        in_specs=[
            pl.BlockSpec((bn, din), lambda i: (i, 0)),
            pl.BlockSpec((din, DP), lambda i: (0, 0)),
            pl.BlockSpec((bn, 1), lambda i: (i, 0)),
            pl.BlockSpec((bn, 1), lambda i: (i, 0)),
        ],
        out_specs=[
            pl.BlockSpec((bn, HALF), lambda i: (i, 0)),
            pl.BlockSpec((bn, HALF), lambda i: (i, 0)),
            pl.BlockSpec((bn, 1), lambda i: (i, 0)),
        ],
        out_shape=[
            jax.ShapeDtypeStruct((n, HALF), jnp.float32),
            jax.ShapeDtypeStruct((n, HALF), jnp.float32),
            jax.ShapeDtypeStruct((n, 1), jnp.float32),
        ],
    )(x, w1p, d0, d1)


def _tc_layer2(al, ar, zl, zr, dinv, b1p, w2p, n, bn):
    grid = n // bn

    def body(al_ref, ar_ref, zl_ref, zr_ref, dinv_ref, b_ref, w_ref,
             ol_ref, or_ref):
        dinv = dinv_ref[...]
        hl = dinv * (al_ref[...] + zl_ref[...])
        hr = dinv * (ar_ref[...] + zr_ref[...])
        h = jnp.concatenate([hl, hr], axis=1) + b_ref[...]
        h = jnp.maximum(h, 0.0)
        z2 = dinv * jnp.dot(h, w_ref[...], preferred_element_type=jnp.float32)
        ol_ref[...] = z2[:, :HALF]
        or_ref[...] = z2[:, HALF:]

    return pl.pallas_call(
        body,
        grid=(grid,),
        in_specs=[
            pl.BlockSpec((bn, HALF), lambda i: (i, 0)),
            pl.BlockSpec((bn, HALF), lambda i: (i, 0)),
            pl.BlockSpec((bn, HALF), lambda i: (i, 0)),
            pl.BlockSpec((bn, HALF), lambda i: (i, 0)),
            pl.BlockSpec((bn, 1), lambda i: (i, 0)),
            pl.BlockSpec((1, DP), lambda i: (0, 0)),
            pl.BlockSpec((DP, DP), lambda i: (0, 0)),
        ],
        out_specs=[
            pl.BlockSpec((bn, HALF), lambda i: (i, 0)),
            pl.BlockSpec((bn, HALF), lambda i: (i, 0)),
        ],
        out_shape=[
            jax.ShapeDtypeStruct((n, HALF), jnp.float32),
            jax.ShapeDtypeStruct((n, HALF), jnp.float32),
        ],
    )(al, ar, zl, zr, dinv, b1p, w2p)


def _tc_final(al, ar, zl, zr, dinv, b2p, batch2d, wlinp, blin2d, n, bn):
    grid = n // bn

    def body(al_ref, ar_ref, zl_ref, zr_ref, dinv_ref, b_ref, bat_ref,
             wlin_ref, blin_ref, out_ref, sums_ref, cnts_ref):
        i = pl.program_id(0)

        @pl.when(i == 0)
        def _():
            sums_ref[...] = jnp.zeros_like(sums_ref)
            cnts_ref[...] = jnp.zeros_like(cnts_ref)

        dinv = dinv_ref[...]
        hl = dinv * (al_ref[...] + zl_ref[...])
        hr = dinv * (ar_ref[...] + zr_ref[...])
        h2 = jnp.concatenate([hl, hr], axis=1) + b_ref[...]
        h2 = jnp.maximum(h2, 0.0)

        gid = lax.broadcasted_iota(jnp.int32, (bn, NUM_GRAPHS), 1)
        m = (bat_ref[...] == gid).astype(jnp.float32)
        sums_ref[...] += lax.dot_general(
            h2, m, (((0,), (0,)), ((), ())),
            preferred_element_type=jnp.float32)
        cnts_ref[...] += jnp.sum(m, axis=0, keepdims=True)

        @pl.when(i == grid - 1)
        def _():
            pooled = sums_ref[...] / jnp.maximum(cnts_ref[...], 1.0)
            out_ref[...] = lax.dot_general(
                wlin_ref[...], pooled, (((0,), (0,)), ((), ())),
                preferred_element_type=jnp.float32) + blin_ref[...]

    return pl.pallas_call(
        body,
        grid=(grid,),
        in_specs=[
            pl.BlockSpec((bn, HALF), lambda i: (i, 0)),
            pl.BlockSpec((bn, HALF), lambda i: (i, 0)),
            pl.BlockSpec((bn, HALF), lambda i: (i, 0)),
            pl.BlockSpec((bn, HALF), lambda i: (i, 0)),
            pl.BlockSpec((bn, 1), lambda i: (i, 0)),
            pl.BlockSpec((1, DP), lambda i: (0, 0)),
            pl.BlockSpec((bn, 1), lambda i: (i, 0)),
            pl.BlockSpec((DP, 1), lambda i: (0, 0)),
            pl.BlockSpec((1, 1), lambda i: (0, 0)),
        ],
        out_specs=pl.BlockSpec((1, NUM_GRAPHS), lambda i: (0, 0)),
        out_shape=jax.ShapeDtypeStruct((1, NUM_GRAPHS), jnp.float32),
        scratch_shapes=[
            pltpu.VMEM((DP, NUM_GRAPHS), jnp.float32),
            pltpu.VMEM((1, NUM_GRAPHS), jnp.float32),
        ],
    )(al, ar, zl, zr, dinv, b2p, batch2d, wlinp, blin2d)


def kernel(x, edge_index, batch, W1, b1, W2, b2, Wlin, blin):
    n = x.shape[0]
    bn = 2000
    dh = W1.shape[1]

    src = edge_index[0]
    dst = edge_index[1]

    w1p = jnp.zeros((x.shape[1], DP), jnp.float32).at[:, :dh].set(W1)
    b1p = jnp.zeros((1, DP), jnp.float32).at[0, :dh].set(b1)
    w2p = jnp.zeros((DP, DP), jnp.float32).at[:dh, :dh].set(W2)
    b2p = jnp.zeros((1, DP), jnp.float32).at[0, :dh].set(b2)
    wlinp = jnp.zeros((DP, 1), jnp.float32).at[:dh].set(Wlin)
    blin2d = blin.reshape(1, 1)

    d0, d1 = _sc_degree(dst, n)
    zl, zr, dinv = _tc_z1(x, w1p, d0.reshape(n, 1), d1.reshape(n, 1), n, bn)
    al, ar = _sc_aggregate(src, dst, zl, zr, n)
    z2l, z2r = _tc_layer2(al, ar, zl, zr, dinv, b1p, w2p, n, bn)
    a2l, a2r = _sc_aggregate(src, dst, z2l, z2r, n)
    out_t = _tc_final(a2l, a2r, z2l, z2r, dinv, b2p, batch.reshape(n, 1),
                      wlinp, blin2d, n, bn)
    return out_t.reshape(NUM_GRAPHS, 1)


# speedup vs baseline: 51.0286x; 1.2130x over previous
import functools

import jax
import jax.numpy as jnp
from jax import lax
from jax.experimental import pallas as pl
from jax.experimental.pallas import tpu as pltpu
from jax.experimental.pallas import tpu_sc as plsc

NUM_GRAPHS = 64
DP = 48
HALF = DP // 2
EB = 1000
NB = 1000
NCORES = 2
NSUB = 16


def _sc_degree(dst, n):
    e = dst.shape[0]
    per_core = e // NCORES
    per_tile = per_core // NSUB
    n_iter = per_tile // EB
    assert per_core * NCORES == e and per_tile * NSUB == per_core
    assert n_iter * EB == per_tile and n % NB == 0
    n_chunks = n // NB
    mesh = plsc.VectorSubcoreMesh(core_axis_name="c", subcore_axis_name="s")

    @functools.partial(
        pl.kernel,
        out_type=[jax.ShapeDtypeStruct((n,), jnp.float32)] * 2,
        mesh=mesh,
        scratch_types=[
            pltpu.VMEM((EB,), jnp.int32),
            pltpu.VMEM((EB,), jnp.float32),
            pltpu.VMEM((NB,), jnp.float32),
            pltpu.VMEM_SHARED((n,), jnp.float32),
        ],
    )
    def k(dst_hbm, out0, out1, idx_v, ones_v, zero_v, acc_sh):
        c = lax.axis_index("c")
        s = lax.axis_index("s")

        @pl.loop(0, EB // 16)
        def _(i):
            ones_v[pl.ds(i * 16, 16)] = jnp.ones((16,), jnp.float32)

        @pl.loop(0, NB // 16)
        def _(i):
            zero_v[pl.ds(i * 16, 16)] = jnp.zeros((16,), jnp.float32)

        for j in range(n_chunks):
            @pl.when(s == j % NSUB)
            def _():
                pltpu.sync_copy(zero_v, acc_sh.at[pl.ds(j * NB, NB)])

        plsc.subcore_barrier()

        base0 = c * per_core + s * per_tile

        @pl.loop(0, n_iter)
        def _(i):
            b = base0 + i * EB
            pltpu.sync_copy(dst_hbm.at[pl.ds(b, EB)], idx_v)
            pltpu.sync_copy(ones_v, acc_sh.at[idx_v], add=True)

        plsc.subcore_barrier()

        for j in range(n_chunks):
            @pl.when(s == j % NSUB)
            def _():
                sl = pl.ds(j * NB, NB)
                pltpu.sync_copy(acc_sh.at[sl], zero_v)

                @pl.when(c == 0)
                def _():
                    pltpu.sync_copy(zero_v, out0.at[sl])

                @pl.when(c == 1)
                def _():
                    pltpu.sync_copy(zero_v, out1.at[sl])

    return k(dst)


def _sc_aggregate(src, dst, z_left, z_right, n):
    e = src.shape[0]
    per_tile = e // NSUB
    n_iter = per_tile // EB
    assert per_tile * NSUB == e and n_iter * EB == per_tile
    n_chunks = n // NB
    mesh = plsc.VectorSubcoreMesh(core_axis_name="c", subcore_axis_name="s")

    assert n_iter % 2 == 0
    half_iter = n_iter // 2

    @functools.partial(
        pl.kernel,
        out_type=[jax.ShapeDtypeStruct((n, HALF), jnp.float32)] * 2,
        mesh=mesh,
        scratch_types=[
            pltpu.VMEM((EB,), jnp.int32),
            pltpu.VMEM((EB,), jnp.int32),
            pltpu.VMEM((EB,), jnp.int32),
            pltpu.VMEM((EB,), jnp.int32),
            pltpu.VMEM((EB, HALF), jnp.float32),
            pltpu.VMEM((EB, HALF), jnp.float32),
            pltpu.VMEM_SHARED((n, HALF), jnp.float32),
            pltpu.SemaphoreType.DMA,
            pltpu.SemaphoreType.DMA,
        ],
        compiler_params=pltpu.CompilerParams(use_tc_tiling_on_sc=False),
    )
    def k(src_hbm, dst_hbm, zl_hbm, zr_hbm, out_l, out_r,
          si0, di0, si1, di1, rows0, rows1, acc_sh, gsem0, gsem1):
        c = lax.axis_index("c")
        s = lax.axis_index("s")
        si = (si0, si1)
        di = (di0, di1)
        rows = (rows0, rows1)
        gsem = (gsem0, gsem1)

        def gather_start(b):
            @pl.when(c == 0)
            def _():
                pltpu.async_copy(zl_hbm.at[si[b]], rows[b], gsem[b])

            @pl.when(c == 1)
            def _():
                pltpu.async_copy(zr_hbm.at[si[b]], rows[b], gsem[b])

        def gather_wait(b):
            @pl.when(c == 0)
            def _():
                pltpu.make_async_copy(zl_hbm.at[si[b]], rows[b], gsem[b]).wait()

            @pl.when(c == 1)
            def _():
                pltpu.make_async_copy(zr_hbm.at[si[b]], rows[b], gsem[b]).wait()

        def load_and_gather(b, base):
            pltpu.sync_copy(src_hbm.at[pl.ds(base, EB)], si[b])
            pltpu.sync_copy(dst_hbm.at[pl.ds(base, EB)], di[b])
            gather_start(b)

        def scatter(b):
            pltpu.sync_copy(rows[b], acc_sh.at[di[b]], add=True)

        @pl.loop(0, EB)
        def _(i):
            rows0[i, pl.ds(0, 16)] = jnp.zeros((16,), jnp.float32)
            rows0[i, pl.ds(HALF - 16, 16)] = jnp.zeros((16,), jnp.float32)

        for j in range(n_chunks):
            @pl.when(s == j % NSUB)
            def _():
                pltpu.sync_copy(rows0, acc_sh.at[pl.ds(j * NB, NB)])

        plsc.subcore_barrier()

        base0 = s * per_tile

        load_and_gather(0, base0)

        @pl.loop(0, half_iter)
        def _(t):
            bA = base0 + (2 * t) * EB

            @pl.when(t + 1 < half_iter)
            def _():
                load_and_gather(1, bA + EB)
                gather_wait(0)
                scatter(0)
                load_and_gather(0, bA + 2 * EB)
                gather_wait(1)
                scatter(1)

            @pl.when(t + 1 == half_iter)
            def _():
                load_and_gather(1, bA + EB)
                gather_wait(0)
                scatter(0)
                gather_wait(1)
                scatter(1)

        plsc.subcore_barrier()

        for j in range(n_chunks):
            @pl.when(s == j % NSUB)
            def _():
                sl = pl.ds(j * NB, NB)
                pltpu.sync_copy(acc_sh.at[sl], rows0)

                @pl.when(c == 0)
                def _():
                    pltpu.sync_copy(rows0, out_l.at[sl])

                @pl.when(c == 1)
                def _():
                    pltpu.sync_copy(rows0, out_r.at[sl])

    return k(src, dst, z_left, z_right)


def _tc_z1(x, w1p, degc, n, bn):
    grid = n // bn
    din = x.shape[1]

    def body(x_ref, w_ref, deg_ref, zl_ref, zr_ref, dinv_ref):
        dinv = lax.rsqrt(1.0 + deg_ref[...])
        xw = jnp.dot(x_ref[...], w_ref[...], preferred_element_type=jnp.float32)
        z = xw * dinv
        zl_ref[...] = z[:, :HALF]
        zr_ref[...] = z[:, HALF:]
        dinv_ref[...] = dinv

    return pl.pallas_call(
        body,
        grid=(grid,),
        in_specs=[
            pl.BlockSpec((bn, din), lambda i: (i, 0)),
            pl.BlockSpec((din, DP), lambda i: (0, 0)),
            pl.BlockSpec((bn, 1), lambda i: (i, 0)),
        ],
        out_specs=[
            pl.BlockSpec((bn, HALF), lambda i: (i, 0)),
            pl.BlockSpec((bn, HALF), lambda i: (i, 0)),
            pl.BlockSpec((bn, 1), lambda i: (i, 0)),
        ],
        out_shape=[
            jax.ShapeDtypeStruct((n, HALF), jnp.float32),
            jax.ShapeDtypeStruct((n, HALF), jnp.float32),
            jax.ShapeDtypeStruct((n, 1), jnp.float32),
        ],
    )(x, w1p, degc)


def _tc_layer2(al, ar, zl, zr, dinv, b1p, w2p, n, bn):
    grid = n // bn

    def body(al_ref, ar_ref, zl_ref, zr_ref, dinv_ref, b_ref, w_ref,
             ol_ref, or_ref):
        dinv = dinv_ref[...]
        hl = dinv * (al_ref[...] + zl_ref[...])
        hr = dinv * (ar_ref[...] + zr_ref[...])
        h = jnp.concatenate([hl, hr], axis=1) + b_ref[...]
        h = jnp.maximum(h, 0.0)
        z2 = dinv * jnp.dot(h, w_ref[...], preferred_element_type=jnp.float32)
        ol_ref[...] = z2[:, :HALF]
        or_ref[...] = z2[:, HALF:]

    return pl.pallas_call(
        body,
        grid=(grid,),
        in_specs=[
            pl.BlockSpec((bn, HALF), lambda i: (i, 0)),
            pl.BlockSpec((bn, HALF), lambda i: (i, 0)),
            pl.BlockSpec((bn, HALF), lambda i: (i, 0)),
            pl.BlockSpec((bn, HALF), lambda i: (i, 0)),
            pl.BlockSpec((bn, 1), lambda i: (i, 0)),
            pl.BlockSpec((1, DP), lambda i: (0, 0)),
            pl.BlockSpec((DP, DP), lambda i: (0, 0)),
        ],
        out_specs=[
            pl.BlockSpec((bn, HALF), lambda i: (i, 0)),
            pl.BlockSpec((bn, HALF), lambda i: (i, 0)),
        ],
        out_shape=[
            jax.ShapeDtypeStruct((n, HALF), jnp.float32),
            jax.ShapeDtypeStruct((n, HALF), jnp.float32),
        ],
    )(al, ar, zl, zr, dinv, b1p, w2p)


def _tc_final(al, ar, zl, zr, dinv, b2p, batch1d, wlinp, blin2d, n, bn):
    grid = n // bn

    def body(al_ref, ar_ref, zl_ref, zr_ref, dinv_ref, b_ref, bat_ref,
             wlin_ref, blin_ref, out_ref, sums_ref, cnts_ref):
        i = pl.program_id(0)

        @pl.when(i == 0)
        def _():
            sums_ref[...] = jnp.zeros_like(sums_ref)
            cnts_ref[...] = jnp.zeros_like(cnts_ref)

        dinv = dinv_ref[...]
        hl = dinv * (al_ref[...] + zl_ref[...])
        hr = dinv * (ar_ref[...] + zr_ref[...])
        h2 = jnp.concatenate([hl, hr], axis=1) + b_ref[...]
        h2 = jnp.maximum(h2, 0.0)

        gid = lax.broadcasted_iota(jnp.int32, (NUM_GRAPHS, bn), 0)
        m = (bat_ref[0] == gid).astype(jnp.float32)
        sums_ref[...] += jnp.dot(m, h2, preferred_element_type=jnp.float32)
        cnts_ref[...] += jnp.sum(m, axis=1, keepdims=True)

        @pl.when(i == grid - 1)
        def _():
            pooled = sums_ref[...] / jnp.maximum(cnts_ref[...], 1.0)
            out_ref[...] = jnp.dot(
                pooled, wlin_ref[...],
                preferred_element_type=jnp.float32) + blin_ref[...]

    return pl.pallas_call(
        body,
        grid=(grid,),
        in_specs=[
            pl.BlockSpec((bn, HALF), lambda i: (i, 0)),
            pl.BlockSpec((bn, HALF), lambda i: (i, 0)),
            pl.BlockSpec((bn, HALF), lambda i: (i, 0)),
            pl.BlockSpec((bn, HALF), lambda i: (i, 0)),
            pl.BlockSpec((bn, 1), lambda i: (i, 0)),
            pl.BlockSpec((1, DP), lambda i: (0, 0)),
            pl.BlockSpec((1, 1, bn), lambda i: (i, 0, 0)),
            pl.BlockSpec((DP, 1), lambda i: (0, 0)),
            pl.BlockSpec((1, 1), lambda i: (0, 0)),
        ],
        out_specs=pl.BlockSpec((NUM_GRAPHS, 1), lambda i: (0, 0)),
        out_shape=jax.ShapeDtypeStruct((NUM_GRAPHS, 1), jnp.float32),
        scratch_shapes=[
            pltpu.VMEM((NUM_GRAPHS, DP), jnp.float32),
            pltpu.VMEM((NUM_GRAPHS, 1), jnp.float32),
        ],
    )(al, ar, zl, zr, dinv, b2p, batch1d, wlinp, blin2d)


def kernel(x, edge_index, batch, W1, b1, W2, b2, Wlin, blin):
    n = x.shape[0]
    bn = 2000
    dh = W1.shape[1]

    src = edge_index[0]
    dst = edge_index[1]

    w1p = jnp.zeros((x.shape[1], DP), jnp.float32).at[:, :dh].set(W1)
    b1p = jnp.zeros((1, DP), jnp.float32).at[0, :dh].set(b1)
    w2p = jnp.zeros((DP, DP), jnp.float32).at[:dh, :dh].set(W2)
    b2p = jnp.zeros((1, DP), jnp.float32).at[0, :dh].set(b2)
    wlinp = jnp.zeros((DP, 1), jnp.float32).at[:dh].set(Wlin)
    blin2d = blin.reshape(1, 1)

    d0, d1 = _sc_degree(dst, n)
    degc = (d0 + d1).reshape(n, 1)
    zl, zr, dinv = _tc_z1(x, w1p, degc, n, bn)
    al, ar = _sc_aggregate(src, dst, zl, zr, n)
    z2l, z2r = _tc_layer2(al, ar, zl, zr, dinv, b1p, w2p, n, bn)
    a2l, a2r = _sc_aggregate(src, dst, z2l, z2r, n)
    return _tc_final(a2l, a2r, z2l, z2r, dinv, b2p,
                     batch.reshape(n // bn, 1, bn), wlinp, blin2d, n, bn)
